# baseline (speedup 1.0000x reference)
import functools

import jax
import jax.numpy as jnp
from jax import lax
from jax.experimental import pallas as pl
from jax.experimental.pallas import tpu as pltpu
from jax.experimental.pallas import tpu_sc as plsc

BATCH = 4096
N_DIMS = 128
NUM_CORES = 2
NUM_SUBCORES = 16
NUM_WORKERS = NUM_CORES * NUM_SUBCORES
ROWS_PER_WORKER = BATCH // NUM_WORKERS
LANES = 16
NUM_CHUNKS = 4
CHUNK = ROWS_PER_WORKER // NUM_CHUNKS


@jax.jit
def _gather_sc(data, indices, start_vec):
    mesh = plsc.VectorSubcoreMesh(core_axis_name="c", subcore_axis_name="s")

    @functools.partial(
        pl.kernel,
        mesh=mesh,
        out_type=jax.ShapeDtypeStruct((BATCH, N_DIMS), jnp.float32),
        scratch_types=[
            pltpu.VMEM((LANES,), jnp.int32),
            pltpu.VMEM((ROWS_PER_WORKER,), jnp.int32),
            pltpu.VMEM((ROWS_PER_WORKER,), jnp.int32),
            pltpu.VMEM((ROWS_PER_WORKER, N_DIMS), jnp.float32),
            pltpu.SemaphoreType.DMA,
            [pltpu.SemaphoreType.DMA] * NUM_CHUNKS,
            [pltpu.SemaphoreType.DMA] * NUM_CHUNKS,
        ],
    )
    def body(data_hbm, idx_hbm, startv_hbm, out_hbm,
             sv_v, pos_v, val_v, rows_v, sem0, gsems, psems):
        wid = lax.axis_index("s") * NUM_CORES + lax.axis_index("c")
        base = wid * ROWS_PER_WORKER
        pltpu.sync_copy(startv_hbm, sv_v)
        s16 = sv_v[...]
        for j in range(ROWS_PER_WORKER // LANES):
            pos_v[pl.ds(j * LANES, LANES)] = s16 + (base + j * LANES)
        pltpu.async_copy(idx_hbm.at[pos_v], val_v, sem0).wait()
        gets = [
            pltpu.async_copy(
                data_hbm.at[val_v.at[pl.ds(c * CHUNK, CHUNK)]],
                rows_v.at[pl.ds(c * CHUNK, CHUNK)],
                gsems[c],
            )
            for c in range(NUM_CHUNKS)
        ]
        puts = []
        for c in range(NUM_CHUNKS):
            gets[c].wait()
            puts.append(
                pltpu.async_copy(
                    rows_v.at[pl.ds(c * CHUNK, CHUNK)],
                    out_hbm.at[pl.ds(base + c * CHUNK, CHUNK)],
                    psems[c],
                )
            )
        for p in puts:
            p.wait()

    return body(data, indices, start_vec)


def kernel(data, indices, index):
    n = indices.shape[0]
    index = jnp.asarray(index, jnp.int32)
    break_condition = index >= n
    new_index = index + BATCH
    start = jnp.clip(index, 0, n - BATCH)
    start_vec = start + jnp.arange(LANES, dtype=jnp.int32)
    batch = _gather_sc(data, indices, start_vec)
    return (batch, new_index, break_condition)


# speedup vs baseline: 1.0381x; 1.0381x over previous
import functools

import jax
import jax.numpy as jnp
from jax import lax
from jax.experimental import pallas as pl
from jax.experimental.pallas import tpu as pltpu
from jax.experimental.pallas import tpu_sc as plsc

BATCH = 4096
N_DIMS = 128
NUM_CORES = 2
NUM_SUBCORES = 16
NUM_WORKERS = NUM_CORES * NUM_SUBCORES
ROWS_PER_WORKER = BATCH // NUM_WORKERS
NUM_CHUNKS = 2
CHUNK = ROWS_PER_WORKER // NUM_CHUNKS


@jax.jit
def _gather_sc(data, indices, positions):
    mesh = plsc.VectorSubcoreMesh(core_axis_name="c", subcore_axis_name="s")

    @functools.partial(
        pl.kernel,
        mesh=mesh,
        out_type=jax.ShapeDtypeStruct((BATCH, N_DIMS), jnp.float32),
        scratch_types=[
            pltpu.VMEM((ROWS_PER_WORKER,), jnp.int32),
            pltpu.VMEM((ROWS_PER_WORKER,), jnp.int32),
            pltpu.VMEM((ROWS_PER_WORKER, N_DIMS), jnp.float32),
            pltpu.SemaphoreType.DMA,
            [pltpu.SemaphoreType.DMA] * NUM_CHUNKS,
            [pltpu.SemaphoreType.DMA] * NUM_CHUNKS,
        ],
    )
    def body(data_hbm, idx_hbm, pos_hbm, out_hbm,
             pos_v, val_v, rows_v, sem0, gsems, psems):
        wid = lax.axis_index("s") * NUM_CORES + lax.axis_index("c")
        base = wid * ROWS_PER_WORKER
        pltpu.sync_copy(pos_hbm.at[pl.ds(base, ROWS_PER_WORKER)], pos_v)
        pltpu.async_copy(idx_hbm.at[pos_v], val_v, sem0).wait()
        gets = [
            pltpu.async_copy(
                data_hbm.at[val_v.at[pl.ds(c * CHUNK, CHUNK)]],
                rows_v.at[pl.ds(c * CHUNK, CHUNK)],
                gsems[c],
            )
            for c in range(NUM_CHUNKS)
        ]
        puts = []
        for c in range(NUM_CHUNKS):
            gets[c].wait()
            puts.append(
                pltpu.async_copy(
                    rows_v.at[pl.ds(c * CHUNK, CHUNK)],
                    out_hbm.at[pl.ds(base + c * CHUNK, CHUNK)],
                    psems[c],
                )
            )
        for p in puts:
            p.wait()

    return body(data, indices, positions)


def kernel(data, indices, index):
    n = indices.shape[0]
    index = jnp.asarray(index, jnp.int32)
    break_condition = index >= n
    new_index = index + BATCH
    start = jnp.clip(index, 0, n - BATCH)
    positions = start + jnp.arange(BATCH, dtype=jnp.int32)
    batch = _gather_sc(data, indices, positions)
    return (batch, new_index, break_condition)
